# Initial kernel scaffold; baseline (speedup 1.0000x reference)
import jax
import jax.numpy as jnp
from jax.experimental import pallas as pl


def kernel(x, target):
    raise NotImplementedError("write your pallas kernel here")


# speedup vs baseline: 1.6297x; 1.6297x over previous
import math

import jax
import jax.numpy as jnp
from jax.experimental import pallas as pl
from jax.experimental.pallas import tpu as pltpu

_SIZE = 100000
_CONF = 0.9
_EPS = float(jnp.float32(0.1 / (_SIZE - 2)))
_H = _CONF * math.log(_CONF) + (_SIZE - 2) * _EPS * math.log(_EPS)
_ROWS_PER_BLOCK = 8


def _body(t_ref, x_ref, o_ref):
    pid = pl.program_id(0)
    t = t_ref[...]
    w = (t != 0).astype(jnp.float32)
    a = _EPS * w
    b = (_CONF - _EPS) * w
    x = x_ref[...]
    iota = jax.lax.broadcasted_iota(jnp.int32, x.shape, 1)
    coeff = jnp.where(iota == t, a + b, a)
    s = jnp.sum(x * coeff)
    s0 = jnp.sum(a * x[:, 0:1])
    contrib = _H * jnp.sum(w) - (s - s0)

    @pl.when(pid == 0)
    def _init():
        o_ref[0, 0] = 0.0

    o_ref[0, 0] += contrib


def kernel(x, target):
    n = x.shape[0]
    r = _ROWS_PER_BLOCK
    t2d = target.astype(jnp.int32).reshape(n, 1)
    out = pl.pallas_call(
        _body,
        grid=(n // r,),
        in_specs=[
            pl.BlockSpec((r, 1), lambda i: (i, 0)),
            pl.BlockSpec((r, _SIZE), lambda i: (i, 0)),
        ],
        out_specs=pl.BlockSpec(memory_space=pltpu.SMEM),
        out_shape=jax.ShapeDtypeStruct((1, 1), jnp.float32),
    )(t2d, x)
    return out[0, 0]
